# baseline (speedup 1.0000x reference)
import functools

import jax
import jax.numpy as jnp
from jax import lax
from jax.experimental import pallas as pl
from jax.experimental.pallas import tpu as pltpu
from jax.experimental.pallas import tpu_sc as plsc

NUM_CORES = 2
NUM_SUBCORES = 16
LANES = 16
NUM_WORKERS = NUM_CORES * NUM_SUBCORES

BATCH = 16384
E_DIM = 64
E_PAD = 128
BPW = BATCH // NUM_WORKERS
NCHUNK = 8
BPC = BPW // NCHUNK
CGROUPS = BPC // LANES


def _score_kernel(sidx_hbm, pidx_hbm, oidx_hbm, ent_hbm, rel_hbm, out_hbm,
                  sidx_v, pidx_v, oidx_v,
                  s_rows0, p_rows0, o_rows0, s_rows1, p_rows1, o_rows1,
                  out_v, sem0, sem1):
    wid = lax.axis_index("s") * NUM_CORES + lax.axis_index("c")
    base = wid * BPW
    lane_iota = lax.iota(jnp.int32, LANES)
    rots = [(lane_iota + r) & (LANES - 1) for r in (8, 4, 2, 1)]

    pltpu.sync_copy(sidx_hbm.at[pl.ds(base, BPW)], sidx_v)
    pltpu.sync_copy(pidx_hbm.at[pl.ds(base, BPW)], pidx_v)
    pltpu.sync_copy(oidx_hbm.at[pl.ds(base, BPW)], oidx_v)

    bufs = [(s_rows0, p_rows0, o_rows0), (s_rows1, p_rows1, o_rows1)]
    sems = [sem0, sem1]

    def issue(c):
        cb = c * BPC
        sb, pb, ob = bufs[c % 2]
        sm = sems[c % 2]
        pltpu.make_async_copy(
            ent_hbm.at[sidx_v.at[pl.ds(cb, BPC)]], sb, sm).start()
        pltpu.make_async_copy(
            rel_hbm.at[pidx_v.at[pl.ds(cb, BPC)]], pb, sm).start()
        pltpu.make_async_copy(
            ent_hbm.at[oidx_v.at[pl.ds(cb, BPC)]], ob, sm).start()

    issue(0)
    for c in range(NCHUNK):
        if c + 1 < NCHUNK:
            issue(c + 1)
        cb = c * BPC
        s_rows, p_rows, o_rows = bufs[c % 2]
        sm = sems[c % 2]
        pltpu.make_async_copy(
            ent_hbm.at[sidx_v.at[pl.ds(cb, BPC)]], s_rows, sm).wait()
        pltpu.make_async_copy(
            rel_hbm.at[pidx_v.at[pl.ds(cb, BPC)]], p_rows, sm).wait()
        pltpu.make_async_copy(
            ent_hbm.at[oidx_v.at[pl.ds(cb, BPC)]], o_rows, sm).wait()

        def group_body(g, carry):
            acc = jnp.zeros((LANES,), jnp.float32)
            for k in range(LANES):
                j = g * LANES + k
                t = jnp.zeros((LANES,), jnp.float32)
                for q in range(E_DIM // LANES):
                    sl = pl.ds(q * LANES, LANES)
                    t = t + s_rows[j, sl] * p_rows[j, sl] * o_rows[j, sl]
                for rv in rots:
                    t = t + t.at[rv].get(mode="promise_in_bounds")
                acc = jnp.where(lane_iota == k, t, acc)
            out_v[pl.ds(cb + g * LANES, LANES)] = 1.0 / (1.0 + jnp.exp(-acc))
            return carry

        lax.fori_loop(0, CGROUPS, group_body, 0)

    pltpu.sync_copy(out_v, out_hbm.at[pl.ds(base, BPW)])


@jax.jit
def _score(s_idx, p_idx, o_idx, ent_table, rel_table):
    mesh = plsc.VectorSubcoreMesh(core_axis_name="c", subcore_axis_name="s")
    run = functools.partial(
        pl.kernel,
        out_type=jax.ShapeDtypeStruct((BATCH,), jnp.float32),
        mesh=mesh,
        compiler_params=pltpu.CompilerParams(
            needs_layout_passes=False, use_tc_tiling_on_sc=False),
        scratch_types=[
            pltpu.VMEM((BPW,), jnp.int32),
            pltpu.VMEM((BPW,), jnp.int32),
            pltpu.VMEM((BPW,), jnp.int32),
            pltpu.VMEM((BPC, E_PAD), jnp.float32),
            pltpu.VMEM((BPC, E_DIM), jnp.float32),
            pltpu.VMEM((BPC, E_PAD), jnp.float32),
            pltpu.VMEM((BPC, E_PAD), jnp.float32),
            pltpu.VMEM((BPC, E_DIM), jnp.float32),
            pltpu.VMEM((BPC, E_PAD), jnp.float32),
            pltpu.VMEM((BPW,), jnp.float32),
            pltpu.SemaphoreType.DMA,
            pltpu.SemaphoreType.DMA,
        ],
    )(_score_kernel)
    return run(s_idx, p_idx, o_idx, ent_table, rel_table)


_TBLK = 32768


def _transpose_body(src_ref, dst_ref):
    dst_ref[:, :E_DIM] = src_ref[...].T


def _to_row_major_padded(table_t):
    d, n = table_t.shape
    grid = (n + _TBLK - 1) // _TBLK
    return pl.pallas_call(
        _transpose_body,
        grid=(grid,),
        in_specs=[pl.BlockSpec((d, _TBLK), lambda i: (0, i))],
        out_specs=pl.BlockSpec((_TBLK, E_PAD), lambda i: (i, 0)),
        out_shape=jax.ShapeDtypeStruct((n, E_PAD), jnp.float32),
    )(table_t)


def kernel(inputs, ent_table, rel_table):
    idx = inputs.astype(jnp.int32)
    s_idx = jnp.bitwise_and(idx[:, 0], 0x7FFFFFFF)
    p_idx = jnp.bitwise_and(idx[:, 1], 0x7FFFFFFF)
    o_idx = jnp.bitwise_and(idx[:, 2], 0x7FFFFFFF)
    ent_rm = _to_row_major_padded(ent_table.T)
    score = _score(s_idx, p_idx, o_idx, ent_rm, rel_table)
    return score[:, None]


# speedup vs baseline: 1.0245x; 1.0245x over previous
import functools

import jax
import jax.numpy as jnp
from jax import lax
from jax.experimental import pallas as pl
from jax.experimental.pallas import tpu as pltpu
from jax.experimental.pallas import tpu_sc as plsc

NUM_CORES = 2
NUM_SUBCORES = 16
LANES = 16
NUM_WORKERS = NUM_CORES * NUM_SUBCORES

BATCH = 16384
E_DIM = 64
E_PAD = 128
BPW = BATCH // NUM_WORKERS
NCHUNK = 4
BPC = BPW // NCHUNK
CGROUPS = BPC // LANES


def _score_kernel(sidx_hbm, pidx_hbm, oidx_hbm, ent_hbm, rel_hbm, out_hbm,
                  sidx_v, pidx_v, oidx_v,
                  s_rows0, p_rows0, o_rows0, s_rows1, p_rows1, o_rows1,
                  out_v, sem0, sem1):
    wid = lax.axis_index("s") * NUM_CORES + lax.axis_index("c")
    base = wid * BPW
    lane_iota = lax.iota(jnp.int32, LANES)
    rots = [(lane_iota + r) & (LANES - 1) for r in (8, 4, 2, 1)]

    pltpu.sync_copy(sidx_hbm.at[pl.ds(base, BPW)], sidx_v)
    pltpu.sync_copy(pidx_hbm.at[pl.ds(base, BPW)], pidx_v)
    pltpu.sync_copy(oidx_hbm.at[pl.ds(base, BPW)], oidx_v)

    bufs = [(s_rows0, p_rows0, o_rows0), (s_rows1, p_rows1, o_rows1)]
    sems = [sem0, sem1]

    def issue(c):
        cb = c * BPC
        sb, pb, ob = bufs[c % 2]
        sm = sems[c % 2]
        pltpu.make_async_copy(
            ent_hbm.at[sidx_v.at[pl.ds(cb, BPC)]], sb, sm).start()
        pltpu.make_async_copy(
            rel_hbm.at[pidx_v.at[pl.ds(cb, BPC)]], pb, sm).start()
        pltpu.make_async_copy(
            ent_hbm.at[oidx_v.at[pl.ds(cb, BPC)]], ob, sm).start()

    issue(0)
    for c in range(NCHUNK):
        if c + 1 < NCHUNK:
            issue(c + 1)
        cb = c * BPC
        s_rows, p_rows, o_rows = bufs[c % 2]
        sm = sems[c % 2]
        pltpu.make_async_copy(
            ent_hbm.at[sidx_v.at[pl.ds(cb, BPC)]], s_rows, sm).wait()
        pltpu.make_async_copy(
            rel_hbm.at[pidx_v.at[pl.ds(cb, BPC)]], p_rows, sm).wait()
        pltpu.make_async_copy(
            ent_hbm.at[oidx_v.at[pl.ds(cb, BPC)]], o_rows, sm).wait()

        def group_body(g, carry):
            acc = jnp.zeros((LANES,), jnp.float32)
            for k in range(LANES):
                j = g * LANES + k
                t = jnp.zeros((LANES,), jnp.float32)
                for q in range(E_DIM // LANES):
                    sl = pl.ds(q * LANES, LANES)
                    t = t + s_rows[j, sl] * p_rows[j, sl] * o_rows[j, sl]
                for rv in rots:
                    t = t + t.at[rv].get(mode="promise_in_bounds")
                acc = jnp.where(lane_iota == k, t, acc)
            out_v[pl.ds(cb + g * LANES, LANES)] = 1.0 / (1.0 + jnp.exp(-acc))
            return carry

        lax.fori_loop(0, CGROUPS, group_body, 0)

    pltpu.sync_copy(out_v, out_hbm.at[pl.ds(base, BPW)])


@jax.jit
def _score(s_idx, p_idx, o_idx, ent_table, rel_table):
    mesh = plsc.VectorSubcoreMesh(core_axis_name="c", subcore_axis_name="s")
    run = functools.partial(
        pl.kernel,
        out_type=jax.ShapeDtypeStruct((BATCH,), jnp.float32),
        mesh=mesh,
        compiler_params=pltpu.CompilerParams(
            needs_layout_passes=False, use_tc_tiling_on_sc=False),
        scratch_types=[
            pltpu.VMEM((BPW,), jnp.int32),
            pltpu.VMEM((BPW,), jnp.int32),
            pltpu.VMEM((BPW,), jnp.int32),
            pltpu.VMEM((BPC, E_PAD), jnp.float32),
            pltpu.VMEM((BPC, E_DIM), jnp.float32),
            pltpu.VMEM((BPC, E_PAD), jnp.float32),
            pltpu.VMEM((BPC, E_PAD), jnp.float32),
            pltpu.VMEM((BPC, E_DIM), jnp.float32),
            pltpu.VMEM((BPC, E_PAD), jnp.float32),
            pltpu.VMEM((BPW,), jnp.float32),
            pltpu.SemaphoreType.DMA,
            pltpu.SemaphoreType.DMA,
        ],
    )(_score_kernel)
    return run(s_idx, p_idx, o_idx, ent_table, rel_table)


_TBLK = 32768


def _transpose_body(src_ref, dst_ref):
    dst_ref[:, :E_DIM] = src_ref[...].T


def _to_row_major_padded(table_t):
    d, n = table_t.shape
    grid = (n + _TBLK - 1) // _TBLK
    return pl.pallas_call(
        _transpose_body,
        grid=(grid,),
        in_specs=[pl.BlockSpec((d, _TBLK), lambda i: (0, i))],
        out_specs=pl.BlockSpec((_TBLK, E_PAD), lambda i: (i, 0)),
        out_shape=jax.ShapeDtypeStruct((n, E_PAD), jnp.float32),
    )(table_t)


def kernel(inputs, ent_table, rel_table):
    idx = inputs.astype(jnp.int32)
    s_idx = jnp.bitwise_and(idx[:, 0], 0x7FFFFFFF)
    p_idx = jnp.bitwise_and(idx[:, 1], 0x7FFFFFFF)
    o_idx = jnp.bitwise_and(idx[:, 2], 0x7FFFFFFF)
    ent_rm = _to_row_major_padded(ent_table.T)
    score = _score(s_idx, p_idx, o_idx, ent_rm, rel_table)
    return score[:, None]
